# Initial kernel scaffold; baseline (speedup 1.0000x reference)
import jax
import jax.numpy as jnp
from jax.experimental import pallas as pl


def kernel(local_features, nodes, edge_index_0, edge_index_1, edge_index_2, edge_index_3, W_agg_0, W_agg_1, W_agg_2, W_agg_3, W1, b1, W2, b2):
    raise NotImplementedError("write your pallas kernel here")


# speedup vs baseline: 5.1097x; 5.1097x over previous
import functools

import jax
import jax.numpy as jnp
from jax import lax
from jax.experimental import pallas as pl
from jax.experimental.pallas import tpu as pltpu
from jax.experimental.pallas import tpu_sc as plsc

N = 10000
E = 320000
FEAT = 128
EMB = 128
R = 4

NUM_CORES = 2
NUM_SUBCORES = 16
CHUNK = 128
CHUNKS_PER_TILE = 157
EPAD = CHUNKS_PER_TILE * CHUNK * NUM_SUBCORES
RELS_PER_CORE = R // NUM_CORES
NACC = 10112
ROWS_PER_TILE = NACC // NUM_SUBCORES
ROW_SIZES = (128, 128, 128, 128, 120)
NPAD = 12288


def _seg_body(lf, src_flat, dst_flat, zeros128, zeros1, ones1,
              seg_o, cnt_o, src_v, dst_v, rows_v, ones_v, acc_sh, cnt_sh, sem):
    c = lax.axis_index("c")
    s = lax.axis_index("s")
    row0 = s * ROWS_PER_TILE
    for j in range(RELS_PER_CORE):
        rel = c * RELS_PER_CORE + j
        pltpu.sync_copy(zeros128, rows_v)
        pltpu.sync_copy(zeros1, ones_v)
        off = 0
        for sz in ROW_SIZES:
            pltpu.sync_copy(rows_v.at[pl.ds(0, sz)],
                            acc_sh.at[pl.ds(row0 + off, sz)])
            pltpu.sync_copy(ones_v.at[pl.ds(0, sz)],
                            cnt_sh.at[pl.ds(row0 + off, sz)])
            off += sz
        pltpu.sync_copy(ones1, ones_v)
        plsc.subcore_barrier()
        ebase = rel * EPAD + s * (CHUNKS_PER_TILE * CHUNK)

        def body(k, carry):
            e0 = ebase + k * CHUNK
            pltpu.sync_copy(src_flat.at[pl.ds(e0, CHUNK)], src_v)
            pltpu.sync_copy(dst_flat.at[pl.ds(e0, CHUNK)], dst_v)
            pltpu.async_copy(lf.at[src_v], rows_v, sem).wait()
            pltpu.sync_copy(rows_v, acc_sh.at[dst_v], add=True)
            pltpu.sync_copy(ones_v, cnt_sh.at[dst_v], add=True)
            return carry

        lax.fori_loop(0, CHUNKS_PER_TILE, body, 0)
        plsc.subcore_barrier()
        orow = rel * NACC + row0
        off = 0
        for sz in ROW_SIZES:
            pltpu.sync_copy(acc_sh.at[pl.ds(row0 + off, sz)],
                            rows_v.at[pl.ds(0, sz)])
            pltpu.sync_copy(rows_v.at[pl.ds(0, sz)],
                            seg_o.at[pl.ds(orow + off, sz)])
            pltpu.sync_copy(cnt_sh.at[pl.ds(row0 + off, sz)],
                            ones_v.at[pl.ds(0, sz)])
            pltpu.sync_copy(ones_v.at[pl.ds(0, sz)],
                            cnt_o.at[pl.ds(orow + off, sz)])
            off += sz
        plsc.subcore_barrier()


_seg_call = functools.partial(
    pl.kernel,
    out_type=(jax.ShapeDtypeStruct((R * NACC, FEAT), jnp.float32),
              jax.ShapeDtypeStruct((R * NACC,), jnp.float32)),
    mesh=plsc.VectorSubcoreMesh(core_axis_name="c", subcore_axis_name="s"),
    scratch_types=[
        pltpu.VMEM((CHUNK,), jnp.int32),
        pltpu.VMEM((CHUNK,), jnp.int32),
        pltpu.VMEM((CHUNK, FEAT), jnp.float32),
        pltpu.VMEM((CHUNK,), jnp.float32),
        pltpu.VMEM_SHARED((NACC, FEAT), jnp.float32),
        pltpu.VMEM_SHARED((NACC,), jnp.float32),
        pltpu.SemaphoreType.DMA,
    ],
)(_seg_body)


def _gather_body(nodes_p, table, out_o, idx_v, rows_v, sem):
    c = lax.axis_index("c")
    s = lax.axis_index("s")
    w = s * NUM_CORES + c
    base = w * (NPAD // (NUM_CORES * NUM_SUBCORES))
    for k in range(NPAD // (NUM_CORES * NUM_SUBCORES) // CHUNK):
        b0 = base + k * CHUNK
        pltpu.sync_copy(nodes_p.at[pl.ds(b0, CHUNK)], idx_v)
        pltpu.async_copy(table.at[idx_v], rows_v, sem).wait()
        pltpu.sync_copy(rows_v, out_o.at[pl.ds(b0, CHUNK)])


_gather_call = functools.partial(
    pl.kernel,
    out_type=jax.ShapeDtypeStruct((NPAD, EMB), jnp.float32),
    mesh=plsc.VectorSubcoreMesh(core_axis_name="c", subcore_axis_name="s"),
    scratch_types=[
        pltpu.VMEM((CHUNK,), jnp.int32),
        pltpu.VMEM((CHUNK, EMB), jnp.float32),
        pltpu.SemaphoreType.DMA,
    ],
)(_gather_body)


ROWS_BLK = 1000


def _mlp_body(lf_r, seg_r, cnt_r, wa_r, w1_r, b1_r, w2_r, b2_r, o_r):
    w1 = w1_r[...]
    inv = 1.0 / jnp.maximum(cnt_r[...], 1.0)
    z = jnp.dot(lf_r[...], w1[:FEAT], preferred_element_type=jnp.float32)
    for r in range(R):
        br = jnp.dot(wa_r[r], w1[FEAT + r * EMB:FEAT + (r + 1) * EMB],
                     preferred_element_type=jnp.float32)
        z = z + jnp.dot(seg_r[r] * inv[:, r:r + 1], br,
                        preferred_element_type=jnp.float32)
    h = jnp.tanh(z + b1_r[...])
    o_r[...] = jnp.dot(h, w2_r[...], preferred_element_type=jnp.float32) + b2_r[...]


_mlp_call = pl.pallas_call(
    _mlp_body,
    grid=(N // ROWS_BLK,),
    in_specs=[
        pl.BlockSpec((ROWS_BLK, FEAT), lambda i: (i, 0)),
        pl.BlockSpec((R, ROWS_BLK, FEAT), lambda i: (0, i, 0)),
        pl.BlockSpec((ROWS_BLK, R), lambda i: (i, 0)),
        pl.BlockSpec((R, FEAT, EMB), lambda i: (0, 0, 0)),
        pl.BlockSpec((FEAT + R * EMB, EMB), lambda i: (0, 0)),
        pl.BlockSpec((1, EMB), lambda i: (0, 0)),
        pl.BlockSpec((EMB, EMB), lambda i: (0, 0)),
        pl.BlockSpec((1, EMB), lambda i: (0, 0)),
    ],
    out_specs=pl.BlockSpec((ROWS_BLK, EMB), lambda i: (i, 0)),
    out_shape=jax.ShapeDtypeStruct((N, EMB), jnp.float32),
)


def kernel(local_features, nodes,
           edge_index_0, edge_index_1, edge_index_2, edge_index_3,
           W_agg_0, W_agg_1, W_agg_2, W_agg_3,
           W1, b1, W2, b2):
    lf = local_features
    src = jnp.stack([edge_index_0[0], edge_index_1[0],
                     edge_index_2[0], edge_index_3[0]]).astype(jnp.int32)
    dst = jnp.stack([edge_index_0[1], edge_index_1[1],
                     edge_index_2[1], edge_index_3[1]]).astype(jnp.int32)
    src_flat = jnp.pad(src, ((0, 0), (0, EPAD - E))).reshape(-1)
    dst_flat = jnp.pad(dst, ((0, 0), (0, EPAD - E)), constant_values=N).reshape(-1)
    zeros128 = jnp.zeros((CHUNK, FEAT), jnp.float32)
    zeros1 = jnp.zeros((CHUNK,), jnp.float32)
    ones1 = jnp.ones((CHUNK,), jnp.float32)

    seg_flat, cnt_flat = _seg_call(lf, src_flat, dst_flat, zeros128, zeros1, ones1)
    seg = seg_flat.reshape(R, NACC, FEAT)[:, :N]
    cnt = cnt_flat.reshape(R, NACC)[:, :N].T

    wa = jnp.stack([W_agg_0, W_agg_1, W_agg_2, W_agg_3])
    O = _mlp_call(lf, seg, cnt, wa, W1, b1.reshape(1, EMB), W2, b2.reshape(1, EMB))

    nodes_p = jnp.pad(nodes.astype(jnp.int32), (0, NPAD - N))
    outp = _gather_call(nodes_p, O)
    return outp[:N]


# speedup vs baseline: 5.3089x; 1.0390x over previous
import functools

import jax
import jax.numpy as jnp
from jax import lax
from jax.experimental import pallas as pl
from jax.experimental.pallas import tpu as pltpu
from jax.experimental.pallas import tpu_sc as plsc

N = 10000
E = 320000
FEAT = 128
EMB = 128
R = 4

NUM_CORES = 2
NUM_SUBCORES = 16
CHUNK = 128
CHUNKS_PER_TILE = 158
EPAD = CHUNKS_PER_TILE * CHUNK * NUM_SUBCORES
RELS_PER_CORE = R // NUM_CORES
NACC = 10112
ROWS_PER_TILE = NACC // NUM_SUBCORES
ROW_SIZES = (128, 128, 128, 128, 120)
NPAD = 12288


def _seg_body(lf, src_flat, dst_flat, zeros128, zeros1, ones1, dummyN,
              seg_o, cnt_o,
              src0, dst0, src1, dst1, rows0, rows1, ones_v, dummy_v,
              acc_sh, cnt_sh, gsem0, gsem1, ssem0, ssem1):
    c = lax.axis_index("c")
    s = lax.axis_index("s")
    row0 = s * ROWS_PER_TILE
    pltpu.sync_copy(dummyN, dummy_v)

    def load_idx(e0, srcb, dstb):
        pltpu.sync_copy(src_flat.at[pl.ds(e0, CHUNK)], srcb)
        pltpu.sync_copy(dst_flat.at[pl.ds(e0, CHUNK)], dstb)

    def fire_gather(srcb, rowsb, gsem):
        pltpu.async_copy(lf.at[srcb], rowsb, gsem)

    def wait_gather(srcb, rowsb, gsem):
        pltpu.make_async_copy(lf.at[srcb], rowsb, gsem).wait()

    def fire_scats(rowsb, dstb, ssem):
        pltpu.async_copy(rowsb, acc_sh.at[dstb], ssem, add=True)
        pltpu.async_copy(ones_v, cnt_sh.at[dstb], ssem, add=True)

    def wait_scats(rowsb, dstb, ssem):
        pltpu.make_async_copy(rowsb, acc_sh.at[dstb], ssem).wait()
        pltpu.make_async_copy(ones_v, cnt_sh.at[dstb], ssem).wait()

    for j in range(RELS_PER_CORE):
        rel = c * RELS_PER_CORE + j
        pltpu.sync_copy(zeros128, rows0)
        pltpu.sync_copy(zeros1, ones_v)
        off = 0
        for sz in ROW_SIZES:
            pltpu.sync_copy(rows0.at[pl.ds(0, sz)],
                            acc_sh.at[pl.ds(row0 + off, sz)])
            pltpu.sync_copy(ones_v.at[pl.ds(0, sz)],
                            cnt_sh.at[pl.ds(row0 + off, sz)])
            off += sz
        pltpu.sync_copy(ones1, ones_v)
        plsc.subcore_barrier()
        ebase = rel * EPAD + s * (CHUNKS_PER_TILE * CHUNK)

        fire_scats(rows0, dummy_v, ssem0)
        fire_scats(rows1, dummy_v, ssem1)
        wait_scats(rows0, dst0, ssem0)
        load_idx(ebase, src0, dst0)
        fire_gather(src0, rows0, gsem0)

        def body(i, carry):
            e1 = ebase + (2 * i + 1) * CHUNK
            wait_scats(rows1, dst1, ssem1)
            load_idx(e1, src1, dst1)
            fire_gather(src1, rows1, gsem1)
            wait_gather(src0, rows0, gsem0)
            fire_scats(rows0, dst0, ssem0)

            e2 = e1 + CHUNK
            wait_scats(rows0, dst0, ssem0)
            load_idx(e2, src0, dst0)
            fire_gather(src0, rows0, gsem0)
            wait_gather(src1, rows1, gsem1)
            fire_scats(rows1, dst1, ssem1)
            return carry

        lax.fori_loop(0, CHUNKS_PER_TILE // 2 - 1, body, 0)
        e_last = ebase + (CHUNKS_PER_TILE - 1) * CHUNK
        wait_scats(rows1, dst1, ssem1)
        load_idx(e_last, src1, dst1)
        fire_gather(src1, rows1, gsem1)
        wait_gather(src0, rows0, gsem0)
        fire_scats(rows0, dst0, ssem0)
        wait_gather(src1, rows1, gsem1)
        fire_scats(rows1, dst1, ssem1)
        wait_scats(rows0, dst0, ssem0)
        wait_scats(rows1, dst1, ssem1)
        plsc.subcore_barrier()

        orow = rel * NACC + row0
        off = 0
        for sz in ROW_SIZES:
            pltpu.sync_copy(acc_sh.at[pl.ds(row0 + off, sz)],
                            rows0.at[pl.ds(0, sz)])
            pltpu.sync_copy(rows0.at[pl.ds(0, sz)],
                            seg_o.at[pl.ds(orow + off, sz)])
            pltpu.sync_copy(cnt_sh.at[pl.ds(row0 + off, sz)],
                            ones_v.at[pl.ds(0, sz)])
            pltpu.sync_copy(ones_v.at[pl.ds(0, sz)],
                            cnt_o.at[pl.ds(orow + off, sz)])
            off += sz
        plsc.subcore_barrier()


_seg_call = functools.partial(
    pl.kernel,
    out_type=(jax.ShapeDtypeStruct((R * NACC, FEAT), jnp.float32),
              jax.ShapeDtypeStruct((R * NACC,), jnp.float32)),
    mesh=plsc.VectorSubcoreMesh(core_axis_name="c", subcore_axis_name="s"),
    scratch_types=[
        pltpu.VMEM((CHUNK,), jnp.int32),
        pltpu.VMEM((CHUNK,), jnp.int32),
        pltpu.VMEM((CHUNK,), jnp.int32),
        pltpu.VMEM((CHUNK,), jnp.int32),
        pltpu.VMEM((CHUNK, FEAT), jnp.float32),
        pltpu.VMEM((CHUNK, FEAT), jnp.float32),
        pltpu.VMEM((CHUNK,), jnp.float32),
        pltpu.VMEM((CHUNK,), jnp.int32),
        pltpu.VMEM_SHARED((NACC, FEAT), jnp.float32),
        pltpu.VMEM_SHARED((NACC,), jnp.float32),
        pltpu.SemaphoreType.DMA,
        pltpu.SemaphoreType.DMA,
        pltpu.SemaphoreType.DMA,
        pltpu.SemaphoreType.DMA,
    ],
)(_seg_body)


def _gather_body(nodes_p, table, out_o, idx_v, rows_v, sem):
    c = lax.axis_index("c")
    s = lax.axis_index("s")
    w = s * NUM_CORES + c
    base = w * (NPAD // (NUM_CORES * NUM_SUBCORES))
    for k in range(NPAD // (NUM_CORES * NUM_SUBCORES) // CHUNK):
        b0 = base + k * CHUNK
        pltpu.sync_copy(nodes_p.at[pl.ds(b0, CHUNK)], idx_v)
        pltpu.async_copy(table.at[idx_v], rows_v, sem).wait()
        pltpu.sync_copy(rows_v, out_o.at[pl.ds(b0, CHUNK)])


_gather_call = functools.partial(
    pl.kernel,
    out_type=jax.ShapeDtypeStruct((NPAD, EMB), jnp.float32),
    mesh=plsc.VectorSubcoreMesh(core_axis_name="c", subcore_axis_name="s"),
    scratch_types=[
        pltpu.VMEM((CHUNK,), jnp.int32),
        pltpu.VMEM((CHUNK, EMB), jnp.float32),
        pltpu.SemaphoreType.DMA,
    ],
)(_gather_body)


ROWS_BLK = 1000


def _mlp_body(lf_r, seg_r, cnt_r, wa_r, w1_r, b1_r, w2_r, b2_r, o_r):
    w1 = w1_r[...]
    inv = 1.0 / jnp.maximum(cnt_r[...], 1.0)
    z = jnp.dot(lf_r[...], w1[:FEAT], preferred_element_type=jnp.float32)
    for r in range(R):
        br = jnp.dot(wa_r[r], w1[FEAT + r * EMB:FEAT + (r + 1) * EMB],
                     preferred_element_type=jnp.float32)
        z = z + jnp.dot(seg_r[r] * inv[:, r:r + 1], br,
                        preferred_element_type=jnp.float32)
    h = jnp.tanh(z + b1_r[...])
    o_r[...] = jnp.dot(h, w2_r[...], preferred_element_type=jnp.float32) + b2_r[...]


_mlp_call = pl.pallas_call(
    _mlp_body,
    grid=(N // ROWS_BLK,),
    in_specs=[
        pl.BlockSpec((ROWS_BLK, FEAT), lambda i: (i, 0)),
        pl.BlockSpec((R, ROWS_BLK, FEAT), lambda i: (0, i, 0)),
        pl.BlockSpec((ROWS_BLK, R), lambda i: (i, 0)),
        pl.BlockSpec((R, FEAT, EMB), lambda i: (0, 0, 0)),
        pl.BlockSpec((FEAT + R * EMB, EMB), lambda i: (0, 0)),
        pl.BlockSpec((1, EMB), lambda i: (0, 0)),
        pl.BlockSpec((EMB, EMB), lambda i: (0, 0)),
        pl.BlockSpec((1, EMB), lambda i: (0, 0)),
    ],
    out_specs=pl.BlockSpec((ROWS_BLK, EMB), lambda i: (i, 0)),
    out_shape=jax.ShapeDtypeStruct((N, EMB), jnp.float32),
)


def kernel(local_features, nodes,
           edge_index_0, edge_index_1, edge_index_2, edge_index_3,
           W_agg_0, W_agg_1, W_agg_2, W_agg_3,
           W1, b1, W2, b2):
    lf = local_features
    src = jnp.stack([edge_index_0[0], edge_index_1[0],
                     edge_index_2[0], edge_index_3[0]]).astype(jnp.int32)
    dst = jnp.stack([edge_index_0[1], edge_index_1[1],
                     edge_index_2[1], edge_index_3[1]]).astype(jnp.int32)
    src_flat = jnp.pad(src, ((0, 0), (0, EPAD - E))).reshape(-1)
    dst_flat = jnp.pad(dst, ((0, 0), (0, EPAD - E)), constant_values=N).reshape(-1)
    zeros128 = jnp.zeros((CHUNK, FEAT), jnp.float32)
    zeros1 = jnp.zeros((CHUNK,), jnp.float32)
    ones1 = jnp.ones((CHUNK,), jnp.float32)
    dummyN = jnp.full((CHUNK,), N, jnp.int32)

    seg_flat, cnt_flat = _seg_call(lf, src_flat, dst_flat, zeros128, zeros1,
                                   ones1, dummyN)
    seg = seg_flat.reshape(R, NACC, FEAT)[:, :N]
    cnt = cnt_flat.reshape(R, NACC)[:, :N].T

    wa = jnp.stack([W_agg_0, W_agg_1, W_agg_2, W_agg_3])
    O = _mlp_call(lf, seg, cnt, wa, W1, b1.reshape(1, EMB), W2, b2.reshape(1, EMB))

    nodes_p = jnp.pad(nodes.astype(jnp.int32), (0, NPAD - N))
    outp = _gather_call(nodes_p, O)
    return outp[:N]
